# baseline (speedup 1.0000x reference)
import jax
import jax.numpy as jnp
from jax import lax
from jax.experimental import pallas as pl
from jax.experimental.pallas import tpu as pltpu
from jax.experimental.pallas import tpu_sc as plsc

N = 10000
E = 320000
D = 128
H = 256

NC = 2
NS = 16
NW = NC * NS
EH = E // 2
EPT = EH // NW
CH = 40
NCHUNK = EPT // CH
CONW = 144
ROWS_PER_TILE = N // NS
ZROWS = 125

_vector_mesh = plsc.VectorSubcoreMesh(core_axis_name="c", subcore_axis_name="s")
_sc_linear = pltpu.CompilerParams(use_tc_tiling_on_sc=False)


def _sc_gather(features, aug16, idx_self, idx_nbr):
    @pl.kernel(
        out_type=[
            jax.ShapeDtypeStruct((EH, D), jnp.float32),
            jax.ShapeDtypeStruct((EH, D), jnp.int32),
        ],
        mesh=_vector_mesh,
        scratch_types=[
            pltpu.VMEM((EPT,), jnp.int32),
            pltpu.VMEM((EPT,), jnp.int32),
            pltpu.VMEM((CH, D), jnp.float32),
            pltpu.VMEM((CH, D), jnp.float32),
            pltpu.VMEM((CH, D), jnp.int32),
            pltpu.VMEM((CH, D), jnp.int32),
            pltpu.SemaphoreType.DMA,
            pltpu.SemaphoreType.DMA,
            pltpu.SemaphoreType.DMA,
            pltpu.SemaphoreType.DMA,
        ],
    )
    def kern(feat_hbm, aug_hbm, idxs_hbm, idxn_hbm, outs_hbm, outn_hbm,
             idxs_v, idxn_v, bufs0, bufs1, bufn0, bufn1,
             gsem0, gsem1, wsem0, wsem1):
        c = lax.axis_index("c")
        s = lax.axis_index("s")
        base = (c * NS + s) * EPT

        i1 = pltpu.async_copy(idxs_hbm.at[pl.ds(base, EPT)], idxs_v, gsem0)
        i2 = pltpu.async_copy(idxn_hbm.at[pl.ds(base, EPT)], idxn_v, gsem1)
        i1.wait()
        i2.wait()

        bufs = (bufs0, bufs1)
        bufn = (bufn0, bufn1)
        gsem = (gsem0, gsem1)
        wsem = (wsem0, wsem1)

        def start_gather(k, slot):
            pltpu.async_copy(
                feat_hbm.at[idxs_v.at[pl.ds(k * CH, CH)]], bufs[slot], gsem[slot])
            pltpu.async_copy(
                aug_hbm.at[idxn_v.at[pl.ds(k * CH, CH)]], bufn[slot], gsem[slot])

        def wait_gather(slot):
            pltpu.make_async_copy(feat_hbm.at[pl.ds(0, CH)], bufs[slot], gsem[slot]).wait()
            pltpu.make_async_copy(aug_hbm.at[pl.ds(0, CH)], bufn[slot], gsem[slot]).wait()

        def start_write(k, slot):
            off = base + k * CH
            pltpu.async_copy(bufs[slot], outs_hbm.at[pl.ds(off, CH)], wsem[slot])
            pltpu.async_copy(bufn[slot], outn_hbm.at[pl.ds(off, CH)], wsem[slot])

        def wait_write(slot):
            pltpu.make_async_copy(feat_hbm.at[pl.ds(0, CH)], bufs[slot], wsem[slot]).wait()
            pltpu.make_async_copy(aug_hbm.at[pl.ds(0, CH)], bufn[slot], wsem[slot]).wait()

        start_gather(0, 0)
        start_gather(1, 1)

        @pl.loop(0, (NCHUNK - 1) // 2)
        def _(j):
            k = j * 2
            wait_gather(0)
            start_write(k, 0)
            wait_gather(1)
            start_write(k + 1, 1)
            wait_write(0)
            start_gather(k + 2, 0)

            @pl.when(k + 3 < NCHUNK)
            def _():
                wait_write(1)
                start_gather(k + 3, 1)

        wait_gather(0)
        start_write(NCHUNK - 1, 0)
        wait_write(1)
        wait_write(0)

    return kern(features, aug16, idx_self, idx_nbr)


def _tc_dense(rows_self, rows_nbr, W1cat, b1cat, W2blk, b2cat, powp):
    BB = 640
    grid = EH // BB

    def body(self_ref, nbr_ref, w1_ref, b1_ref, w2_ref, b2_ref, p_ref,
             em_ref, den_ref):
        packed = nbr_ref[...]
        lo = jax.lax.bitcast_convert_type(
            jnp.left_shift(packed, 16), jnp.float32)
        hi = jax.lax.bitcast_convert_type(
            jnp.bitwise_and(packed, jnp.int32(-65536)), jnp.float32)
        x = jnp.concatenate(
            [self_ref[...].astype(jnp.bfloat16),
             lo.astype(jnp.bfloat16), hi.astype(jnp.bfloat16)], axis=1)
        h = jnp.dot(x, w1_ref[...], preferred_element_type=jnp.float32)
        h = h + b1_ref[...]
        h = jnp.where(h > 0, h, 0.01 * h)
        gm = jnp.dot(h.astype(jnp.bfloat16), w2_ref[...],
                     preferred_element_type=jnp.float32)
        gm = gm + b2_ref[...]
        g = gm[:, 0:1]
        m = gm[:, 1:129]
        w = lo[:, 64:65] + hi[:, 64:65]
        p = p_ref[0, 0]
        e = (w ** p) * jnp.exp(g)
        em_ref[...] = e * m
        den_ref[...] = jnp.concatenate(
            [e, jnp.zeros((BB, 15), jnp.float32)], axis=1)

    return pl.pallas_call(
        body,
        grid=(grid,),
        in_specs=[
            pl.BlockSpec((BB, D), lambda i: (i, 0)),
            pl.BlockSpec((BB, D), lambda i: (i, 0)),
            pl.BlockSpec((3 * D, 2 * H), lambda i: (0, 0)),
            pl.BlockSpec((1, 2 * H), lambda i: (0, 0)),
            pl.BlockSpec((2 * H, D + 1), lambda i: (0, 0)),
            pl.BlockSpec((1, D + 1), lambda i: (0, 0)),
            pl.BlockSpec((1, 1), lambda i: (0, 0)),
        ],
        out_specs=[
            pl.BlockSpec((BB, D), lambda i: (i, 0)),
            pl.BlockSpec((BB, 16), lambda i: (i, 0)),
        ],
        out_shape=[
            jax.ShapeDtypeStruct((EH, D), jnp.float32),
            jax.ShapeDtypeStruct((EH, 16), jnp.float32),
        ],
    )(rows_self, rows_nbr, W1cat, b1cat, W2blk, b2cat, powp)


def _sc_scatter(em, e16, idx3):
    @pl.kernel(
        out_type=[
            jax.ShapeDtypeStruct((NC, N, D), jnp.float32),
            jax.ShapeDtypeStruct((NC, N, 16), jnp.float32),
        ],
        mesh=_vector_mesh,
        scratch_types=[
            pltpu.VMEM_SHARED((N, D), jnp.float32),
            pltpu.VMEM_SHARED((N, 16), jnp.float32),
            pltpu.VMEM((ZROWS, D), jnp.float32),
            pltpu.VMEM((ZROWS, 16), jnp.float32),
            pltpu.VMEM((NCHUNK, CH), jnp.int32),
            pltpu.VMEM((CH, D), jnp.float32),
            pltpu.VMEM((CH, D), jnp.float32),
            pltpu.VMEM((CH, 16), jnp.float32),
            pltpu.VMEM((CH, 16), jnp.float32),
            pltpu.SemaphoreType.DMA,
            pltpu.SemaphoreType.DMA,
        ],
        compiler_params=_sc_linear,
    )
    def kern(em_hbm, e16_hbm, idx_hbm, out_em, out_den,
             shared_em, shared_den, zbuf, zbuf16, idx_v,
             cbuf0, cbuf1, dbuf0, dbuf1, lsem0, lsem1):
        c = lax.axis_index("c")
        s = lax.axis_index("s")
        wid = c * NS + s

        pltpu.async_copy(idx_hbm.at[wid], idx_v, lsem0).wait()

        @pl.loop(0, ZROWS)
        def _(r):
            @pl.loop(0, D // 16)
            def _(ct):
                zbuf[r, pl.ds(ct * 16, 16)] = jnp.zeros((16,), jnp.float32)
            zbuf16[r, :] = jnp.zeros((16,), jnp.float32)

        @pl.loop(0, ROWS_PER_TILE // ZROWS)
        def _(j):
            row = s * ROWS_PER_TILE + j * ZROWS
            pltpu.sync_copy(zbuf, shared_em.at[pl.ds(row, ZROWS)])
            pltpu.sync_copy(zbuf16, shared_den.at[pl.ds(row, ZROWS)])

        plsc.subcore_barrier()

        base = wid * EPT
        cbuf = (cbuf0, cbuf1)
        dbuf = (dbuf0, dbuf1)
        lsem = (lsem0, lsem1)

        def start_load(k, slot):
            off = base + k * CH
            pltpu.async_copy(em_hbm.at[pl.ds(off, CH)], cbuf[slot], lsem[slot])
            pltpu.async_copy(e16_hbm.at[pl.ds(off, CH)], dbuf[slot], lsem[slot])

        def wait_load(slot):
            pltpu.make_async_copy(em_hbm.at[pl.ds(0, CH)], cbuf[slot], lsem[slot]).wait()
            pltpu.make_async_copy(e16_hbm.at[pl.ds(0, CH)], dbuf[slot], lsem[slot]).wait()

        def add_streams(k, slot):
            pltpu.sync_copy(cbuf[slot], shared_em.at[idx_v.at[k]], add=True)
            pltpu.sync_copy(dbuf[slot], shared_den.at[idx_v.at[k]], add=True)

        start_load(0, 0)
        start_load(1, 1)

        @pl.loop(0, (NCHUNK - 1) // 2)
        def _(j):
            k = j * 2
            wait_load(0)
            add_streams(k, 0)
            start_load(k + 2, 0)
            wait_load(1)
            add_streams(k + 1, 1)

            @pl.when(k + 3 < NCHUNK)
            def _():
                start_load(k + 3, 1)

        wait_load(0)
        add_streams(NCHUNK - 1, 0)

        plsc.subcore_barrier()

        @pl.loop(0, ROWS_PER_TILE // ZROWS)
        def _(j):
            row = s * ROWS_PER_TILE + j * ZROWS
            pltpu.sync_copy(shared_em.at[pl.ds(row, ZROWS)],
                            out_em.at[c].at[pl.ds(row, ZROWS)])
            pltpu.sync_copy(shared_den.at[pl.ds(row, ZROWS)],
                            out_den.at[c].at[pl.ds(row, ZROWS)])

    return kern(em, e16, idx3)


def _tc_finalize(em1, den1, em2, den2, features):
    BN = 2000

    def body(e1_ref, d1_ref, e2_ref, d2_ref, f_ref, o_ref):
        num = (e1_ref[0] + e1_ref[1] + e2_ref[0] + e2_ref[1])
        den = (d1_ref[0, :, 0:1] + d1_ref[1, :, 0:1]
               + d2_ref[0, :, 0:1] + d2_ref[1, :, 0:1])
        o_ref[...] = num / (den + 1e-10) + f_ref[...]

    return pl.pallas_call(
        body,
        grid=(N // BN,),
        in_specs=[
            pl.BlockSpec((NC, BN, D), lambda i: (0, i, 0)),
            pl.BlockSpec((NC, BN, 16), lambda i: (0, i, 0)),
            pl.BlockSpec((NC, BN, D), lambda i: (0, i, 0)),
            pl.BlockSpec((NC, BN, 16), lambda i: (0, i, 0)),
            pl.BlockSpec((BN, D), lambda i: (i, 0)),
        ],
        out_specs=pl.BlockSpec((BN, D), lambda i: (i, 0)),
        out_shape=jax.ShapeDtypeStruct((N, D), jnp.float32),
    )(em1, den1, em2, den2, features)


def kernel(node_weights, node_prev_features, self_idx, neighbor_idx,
           gate_W1, gate_b1, gate_W2, gate_b2,
           msg_W1, msg_b1, msg_W2, msg_b2, pow_param):
    idx_self = self_idx.astype(jnp.int32)
    idx_nbr = neighbor_idx.astype(jnp.int32)
    feats = node_prev_features.astype(jnp.float32)

    w32 = node_weights.astype(jnp.float32)
    w_hi = w32.astype(jnp.bfloat16)
    w_lo = (w32 - w_hi.astype(jnp.float32)).astype(jnp.bfloat16)
    feats16 = feats.astype(jnp.bfloat16)
    ev = jax.lax.bitcast_convert_type(feats16[:, 0::2], jnp.uint16).astype(jnp.uint32)
    od = jax.lax.bitcast_convert_type(feats16[:, 1::2], jnp.uint16).astype(jnp.uint32)
    wword = (jax.lax.bitcast_convert_type(w_hi, jnp.uint16).astype(jnp.uint32)
             | (jax.lax.bitcast_convert_type(w_lo, jnp.uint16).astype(jnp.uint32) << 16))
    packed = jnp.concatenate(
        [ev | (od << 16), wword, jnp.zeros((N, D - 65), jnp.uint32)], axis=1)
    nbr_packed = jax.lax.bitcast_convert_type(packed, jnp.int32)

    base = jnp.concatenate([gate_W1, msg_W1], axis=1)
    nbr_rows = base[D:]
    W1cat = jnp.concatenate(
        [base[:D],
         nbr_rows[0::2], jnp.zeros((64, 2 * H), jnp.float32),
         nbr_rows[1::2], jnp.zeros((64, 2 * H), jnp.float32)],
        axis=0).astype(jnp.bfloat16)
    b1cat = jnp.concatenate([gate_b1, msg_b1])[None, :].astype(jnp.float32)
    W2blk = jnp.zeros((2 * H, D + 1), jnp.float32)
    W2blk = W2blk.at[:H, 0:1].set(gate_W2)
    W2blk = W2blk.at[H:, 1:].set(msg_W2)
    W2blk = W2blk.astype(jnp.bfloat16)
    b2cat = jnp.concatenate([gate_b2, msg_b2])[None, :].astype(jnp.float32)
    powp = pow_param.reshape(1, 1).astype(jnp.float32)

    partials = []
    for lo_e in (0, EH):
        ids = lax.dynamic_slice_in_dim(idx_self, lo_e, EH)
        idn = lax.dynamic_slice_in_dim(idx_nbr, lo_e, EH)
        rows_self, rows_nbr = _sc_gather(feats, nbr_packed, ids, idn)
        em, e16 = _tc_dense(rows_self, rows_nbr, W1cat, b1cat, W2blk, b2cat, powp)
        partials.append(_sc_scatter(em, e16, ids.reshape(NW, NCHUNK, CH)))
    return _tc_finalize(partials[0][0], partials[0][1],
                        partials[1][0], partials[1][1], feats)


# speedup vs baseline: 1.0618x; 1.0618x over previous
import jax
import jax.numpy as jnp
from jax import lax
from jax.experimental import pallas as pl
from jax.experimental.pallas import tpu as pltpu
from jax.experimental.pallas import tpu_sc as plsc

N = 10000
E = 320000
D = 128
H = 256

NC = 2
NS = 16
NW = NC * NS
EH = E // 2
EPT = EH // NW
CH = 40
NCHUNK = EPT // CH
ROWS_PER_TILE = N // NS
ZROWS = 125

_vector_mesh = plsc.VectorSubcoreMesh(core_axis_name="c", subcore_axis_name="s")
_sc_linear = pltpu.CompilerParams(use_tc_tiling_on_sc=False)


def _sc_gather(features, aug16, idx_self, idx_nbr):
    @pl.kernel(
        out_type=[
            jax.ShapeDtypeStruct((EH, D), jnp.float32),
            jax.ShapeDtypeStruct((EH, D), jnp.int32),
        ],
        mesh=_vector_mesh,
        scratch_types=[
            pltpu.VMEM((EPT,), jnp.int32),
            pltpu.VMEM((EPT,), jnp.int32),
            pltpu.VMEM((80, D), jnp.float32),
            pltpu.VMEM((80, D), jnp.float32),
            pltpu.VMEM((80, D), jnp.int32),
            pltpu.VMEM((80, D), jnp.int32),
            pltpu.SemaphoreType.DMA,
            pltpu.SemaphoreType.DMA,
            pltpu.SemaphoreType.DMA,
            pltpu.SemaphoreType.DMA,
        ],
    )
    def kern(feat_hbm, aug_hbm, idxs_hbm, idxn_hbm, outs_hbm, outn_hbm,
             idxs_v, idxn_v, bufs0, bufs1, bufn0, bufn1,
             gsem0, gsem1, wsem0, wsem1):
        c = lax.axis_index("c")
        s = lax.axis_index("s")
        base = (c * NS + s) * EPT

        i1 = pltpu.async_copy(idxs_hbm.at[pl.ds(base, EPT)], idxs_v, gsem0)
        i2 = pltpu.async_copy(idxn_hbm.at[pl.ds(base, EPT)], idxn_v, gsem1)
        i1.wait()
        i2.wait()

        bufs = (bufs0, bufs1)
        bufn = (bufn0, bufn1)
        gsem = (gsem0, gsem1)
        wsem = (wsem0, wsem1)

        GCH = 80
        NFULL = 62

        def start_gather(k, slot):
            pltpu.async_copy(
                feat_hbm.at[idxs_v.at[pl.ds(k * GCH, GCH)]], bufs[slot], gsem[slot])
            pltpu.async_copy(
                aug_hbm.at[idxn_v.at[pl.ds(k * GCH, GCH)]], bufn[slot], gsem[slot])

        def wait_gather(slot):
            pltpu.make_async_copy(feat_hbm.at[pl.ds(0, GCH)], bufs[slot], gsem[slot]).wait()
            pltpu.make_async_copy(aug_hbm.at[pl.ds(0, GCH)], bufn[slot], gsem[slot]).wait()

        def start_write(k, slot):
            off = base + k * GCH
            pltpu.async_copy(bufs[slot], outs_hbm.at[pl.ds(off, GCH)], wsem[slot])
            pltpu.async_copy(bufn[slot], outn_hbm.at[pl.ds(off, GCH)], wsem[slot])

        def wait_write(slot):
            pltpu.make_async_copy(feat_hbm.at[pl.ds(0, GCH)], bufs[slot], wsem[slot]).wait()
            pltpu.make_async_copy(aug_hbm.at[pl.ds(0, GCH)], bufn[slot], wsem[slot]).wait()

        start_gather(0, 0)
        start_gather(1, 1)

        @pl.loop(0, NFULL // 2)
        def _(j):
            k = j * 2
            wait_gather(0)
            start_write(k, 0)
            wait_gather(1)
            start_write(k + 1, 1)
            wait_write(0)

            @pl.when(k + 2 < NFULL)
            def _():
                start_gather(k + 2, 0)

            wait_write(1)

            @pl.when(k + 3 < NFULL)
            def _():
                start_gather(k + 3, 1)

        toff = NFULL * GCH
        pltpu.async_copy(
            feat_hbm.at[idxs_v.at[pl.ds(toff, 40)]],
            bufs0.at[pl.ds(0, 40)], gsem0)
        pltpu.async_copy(
            aug_hbm.at[idxn_v.at[pl.ds(toff, 40)]],
            bufn0.at[pl.ds(0, 40)], gsem0)
        pltpu.make_async_copy(feat_hbm.at[pl.ds(0, 40)],
                              bufs0.at[pl.ds(0, 40)], gsem0).wait()
        pltpu.make_async_copy(aug_hbm.at[pl.ds(0, 40)],
                              bufn0.at[pl.ds(0, 40)], gsem0).wait()
        pltpu.sync_copy(bufs0.at[pl.ds(0, 40)],
                        outs_hbm.at[pl.ds(base + toff, 40)])
        pltpu.sync_copy(bufn0.at[pl.ds(0, 40)],
                        outn_hbm.at[pl.ds(base + toff, 40)])

    return kern(features, aug16, idx_self, idx_nbr)


def _tc_dense(rows_self, rows_nbr, W1cat, b1cat, W2blk, b2cat, powp):
    BB = 640
    grid = EH // BB

    def body(self_ref, nbr_ref, w1_ref, b1_ref, w2_ref, b2_ref, p_ref,
             em_ref, den_ref):
        packed = nbr_ref[...]
        lo = jax.lax.bitcast_convert_type(
            jnp.left_shift(packed, 16), jnp.float32)
        hi = jax.lax.bitcast_convert_type(
            jnp.bitwise_and(packed, jnp.int32(-65536)), jnp.float32)
        x = jnp.concatenate(
            [self_ref[...].astype(jnp.bfloat16),
             lo.astype(jnp.bfloat16), hi.astype(jnp.bfloat16)], axis=1)
        h = jnp.dot(x, w1_ref[...], preferred_element_type=jnp.float32)
        h = h + b1_ref[...]
        h = jnp.where(h > 0, h, 0.01 * h)
        gm = jnp.dot(h.astype(jnp.bfloat16), w2_ref[...],
                     preferred_element_type=jnp.float32)
        gm = gm + b2_ref[...]
        g = gm[:, 0:1]
        m = gm[:, 1:129]
        w = lo[:, 64:65] + hi[:, 64:65]
        p = p_ref[0, 0]
        e = (w ** p) * jnp.exp(g)
        em_ref[...] = e * m
        den_ref[...] = jnp.concatenate(
            [e, jnp.zeros((BB, 15), jnp.float32)], axis=1)

    return pl.pallas_call(
        body,
        grid=(grid,),
        in_specs=[
            pl.BlockSpec((BB, D), lambda i: (i, 0)),
            pl.BlockSpec((BB, D), lambda i: (i, 0)),
            pl.BlockSpec((3 * D, 2 * H), lambda i: (0, 0)),
            pl.BlockSpec((1, 2 * H), lambda i: (0, 0)),
            pl.BlockSpec((2 * H, D + 1), lambda i: (0, 0)),
            pl.BlockSpec((1, D + 1), lambda i: (0, 0)),
            pl.BlockSpec((1, 1), lambda i: (0, 0)),
        ],
        out_specs=[
            pl.BlockSpec((BB, D), lambda i: (i, 0)),
            pl.BlockSpec((BB, 16), lambda i: (i, 0)),
        ],
        out_shape=[
            jax.ShapeDtypeStruct((EH, D), jnp.float32),
            jax.ShapeDtypeStruct((EH, 16), jnp.float32),
        ],
    )(rows_self, rows_nbr, W1cat, b1cat, W2blk, b2cat, powp)


def _sc_scatter(em, e16, idx3):
    @pl.kernel(
        out_type=[
            jax.ShapeDtypeStruct((NC, N, D), jnp.float32),
            jax.ShapeDtypeStruct((NC, N, 16), jnp.float32),
        ],
        mesh=_vector_mesh,
        scratch_types=[
            pltpu.VMEM_SHARED((N, D), jnp.float32),
            pltpu.VMEM_SHARED((N, 16), jnp.float32),
            pltpu.VMEM((ZROWS, D), jnp.float32),
            pltpu.VMEM((ZROWS, 16), jnp.float32),
            pltpu.VMEM((NCHUNK, CH), jnp.int32),
            pltpu.VMEM((CH, D), jnp.float32),
            pltpu.VMEM((CH, D), jnp.float32),
            pltpu.VMEM((CH, 16), jnp.float32),
            pltpu.VMEM((CH, 16), jnp.float32),
            pltpu.SemaphoreType.DMA,
            pltpu.SemaphoreType.DMA,
        ],
        compiler_params=_sc_linear,
    )
    def kern(em_hbm, e16_hbm, idx_hbm, out_em, out_den,
             shared_em, shared_den, zbuf, zbuf16, idx_v,
             cbuf0, cbuf1, dbuf0, dbuf1, lsem0, lsem1):
        c = lax.axis_index("c")
        s = lax.axis_index("s")
        wid = c * NS + s

        pltpu.async_copy(idx_hbm.at[wid], idx_v, lsem0).wait()

        @pl.loop(0, ZROWS)
        def _(r):
            @pl.loop(0, D // 16)
            def _(ct):
                zbuf[r, pl.ds(ct * 16, 16)] = jnp.zeros((16,), jnp.float32)
            zbuf16[r, :] = jnp.zeros((16,), jnp.float32)

        @pl.loop(0, ROWS_PER_TILE // ZROWS)
        def _(j):
            row = s * ROWS_PER_TILE + j * ZROWS
            pltpu.sync_copy(zbuf, shared_em.at[pl.ds(row, ZROWS)])
            pltpu.sync_copy(zbuf16, shared_den.at[pl.ds(row, ZROWS)])

        plsc.subcore_barrier()

        base = wid * EPT
        cbuf = (cbuf0, cbuf1)
        dbuf = (dbuf0, dbuf1)
        lsem = (lsem0, lsem1)

        def start_load(k, slot):
            off = base + k * CH
            pltpu.async_copy(em_hbm.at[pl.ds(off, CH)], cbuf[slot], lsem[slot])
            pltpu.async_copy(e16_hbm.at[pl.ds(off, CH)], dbuf[slot], lsem[slot])

        def wait_load(slot):
            pltpu.make_async_copy(em_hbm.at[pl.ds(0, CH)], cbuf[slot], lsem[slot]).wait()
            pltpu.make_async_copy(e16_hbm.at[pl.ds(0, CH)], dbuf[slot], lsem[slot]).wait()

        def add_streams(k, slot):
            pltpu.sync_copy(cbuf[slot], shared_em.at[idx_v.at[k]], add=True)
            pltpu.sync_copy(dbuf[slot], shared_den.at[idx_v.at[k]], add=True)

        start_load(0, 0)
        start_load(1, 1)

        @pl.loop(0, (NCHUNK - 1) // 2)
        def _(j):
            k = j * 2
            wait_load(0)
            add_streams(k, 0)
            start_load(k + 2, 0)
            wait_load(1)
            add_streams(k + 1, 1)

            @pl.when(k + 3 < NCHUNK)
            def _():
                start_load(k + 3, 1)

        wait_load(0)
        add_streams(NCHUNK - 1, 0)

        plsc.subcore_barrier()

        @pl.loop(0, ROWS_PER_TILE // ZROWS)
        def _(j):
            row = s * ROWS_PER_TILE + j * ZROWS
            pltpu.sync_copy(shared_em.at[pl.ds(row, ZROWS)],
                            out_em.at[c].at[pl.ds(row, ZROWS)])
            pltpu.sync_copy(shared_den.at[pl.ds(row, ZROWS)],
                            out_den.at[c].at[pl.ds(row, ZROWS)])

    return kern(em, e16, idx3)


def _tc_finalize(em1, den1, em2, den2, features):
    BN = 2000

    def body(e1_ref, d1_ref, e2_ref, d2_ref, f_ref, o_ref):
        num = (e1_ref[0] + e1_ref[1] + e2_ref[0] + e2_ref[1])
        den = (d1_ref[0, :, 0:1] + d1_ref[1, :, 0:1]
               + d2_ref[0, :, 0:1] + d2_ref[1, :, 0:1])
        o_ref[...] = num / (den + 1e-10) + f_ref[...]

    return pl.pallas_call(
        body,
        grid=(N // BN,),
        in_specs=[
            pl.BlockSpec((NC, BN, D), lambda i: (0, i, 0)),
            pl.BlockSpec((NC, BN, 16), lambda i: (0, i, 0)),
            pl.BlockSpec((NC, BN, D), lambda i: (0, i, 0)),
            pl.BlockSpec((NC, BN, 16), lambda i: (0, i, 0)),
            pl.BlockSpec((BN, D), lambda i: (i, 0)),
        ],
        out_specs=pl.BlockSpec((BN, D), lambda i: (i, 0)),
        out_shape=jax.ShapeDtypeStruct((N, D), jnp.float32),
    )(em1, den1, em2, den2, features)


def kernel(node_weights, node_prev_features, self_idx, neighbor_idx,
           gate_W1, gate_b1, gate_W2, gate_b2,
           msg_W1, msg_b1, msg_W2, msg_b2, pow_param):
    idx_self = self_idx.astype(jnp.int32)
    idx_nbr = neighbor_idx.astype(jnp.int32)
    feats = node_prev_features.astype(jnp.float32)

    w32 = node_weights.astype(jnp.float32)
    w_hi = w32.astype(jnp.bfloat16)
    w_lo = (w32 - w_hi.astype(jnp.float32)).astype(jnp.bfloat16)
    feats16 = feats.astype(jnp.bfloat16)
    ev = jax.lax.bitcast_convert_type(feats16[:, 0::2], jnp.uint16).astype(jnp.uint32)
    od = jax.lax.bitcast_convert_type(feats16[:, 1::2], jnp.uint16).astype(jnp.uint32)
    wword = (jax.lax.bitcast_convert_type(w_hi, jnp.uint16).astype(jnp.uint32)
             | (jax.lax.bitcast_convert_type(w_lo, jnp.uint16).astype(jnp.uint32) << 16))
    packed = jnp.concatenate(
        [ev | (od << 16), wword, jnp.zeros((N, D - 65), jnp.uint32)], axis=1)
    nbr_packed = jax.lax.bitcast_convert_type(packed, jnp.int32)

    base = jnp.concatenate([gate_W1, msg_W1], axis=1)
    nbr_rows = base[D:]
    W1cat = jnp.concatenate(
        [base[:D],
         nbr_rows[0::2], jnp.zeros((64, 2 * H), jnp.float32),
         nbr_rows[1::2], jnp.zeros((64, 2 * H), jnp.float32)],
        axis=0).astype(jnp.bfloat16)
    b1cat = jnp.concatenate([gate_b1, msg_b1])[None, :].astype(jnp.float32)
    W2blk = jnp.zeros((2 * H, D + 1), jnp.float32)
    W2blk = W2blk.at[:H, 0:1].set(gate_W2)
    W2blk = W2blk.at[H:, 1:].set(msg_W2)
    W2blk = W2blk.astype(jnp.bfloat16)
    b2cat = jnp.concatenate([gate_b2, msg_b2])[None, :].astype(jnp.float32)
    powp = pow_param.reshape(1, 1).astype(jnp.float32)

    partials = []
    for lo_e in (0, EH):
        ids = lax.dynamic_slice_in_dim(idx_self, lo_e, EH)
        idn = lax.dynamic_slice_in_dim(idx_nbr, lo_e, EH)
        rows_self, rows_nbr = _sc_gather(feats, nbr_packed, ids, idn)
        em, e16 = _tc_dense(rows_self, rows_nbr, W1cat, b1cat, W2blk, b2cat, powp)
        partials.append(_sc_scatter(em, e16, ids.reshape(NW, NCHUNK, CH)))
    return _tc_finalize(partials[0][0], partials[0][1],
                        partials[1][0], partials[1][1], feats)
